# baseline (speedup 1.0000x reference)
import functools

import jax
import jax.numpy as jnp
from jax import lax
from jax.experimental import pallas as pl
from jax.experimental.pallas import tpu as pltpu
from jax.experimental.pallas import tpu_sc as plsc

TB = 128
SC_NC = 2
SC_NS = 16


def _router_body(h_ref, rw_ref, pwt_ref, bexp_ref, pos_ref, wc_ref,
                 *, T, E, TB_, NBMAX):
    f32 = jnp.float32
    bf16 = jnp.bfloat16
    h = h_ref[...]
    rw = rw_ref[...]
    logits = jax.lax.dot_general(h.astype(bf16), rw.astype(bf16),
                                 (((1,), (1,)), ((), ())),
                                 preferred_element_type=f32)
    lane = jax.lax.broadcasted_iota(jnp.int32, (T, E), 1).astype(f32)
    m1 = jnp.max(logits, axis=1, keepdims=True)
    i1 = jnp.min(jnp.where(logits == m1, lane, float(E)), axis=1, keepdims=True)
    e0 = (lane == i1).astype(f32)
    neg = jnp.where(lane == i1, -jnp.inf, logits)
    m2 = jnp.max(neg, axis=1, keepdims=True)
    i2 = jnp.min(jnp.where(neg == m2, lane, float(E)), axis=1, keepdims=True)
    e1 = (lane == i2).astype(f32)
    w0 = 1.0 / (1.0 + jnp.exp(m2 - m1))
    w1 = 1.0 - w0

    ones_col = jnp.ones((T, 1), f32)
    ecnt = jax.lax.dot_general(e0 + e1, ones_col, (((0,), (0,)), ((), ())),
                               preferred_element_type=f32)
    nblk = jnp.floor((ecnt + (TB_ - 1)) / TB_)
    tri = (jax.lax.broadcasted_iota(jnp.int32, (E, E), 1)
           < jax.lax.broadcasted_iota(jnp.int32, (E, E), 0)).astype(f32)
    excl = jax.lax.dot_general(tri, nblk, (((1,), (0,)), ((), ())),
                               preferred_element_type=f32)
    incl = excl + nblk

    Epairs = e0 + e1
    RB = 256
    c_blocks = []
    for tb in range(T // RB):
        r = jax.lax.broadcasted_iota(jnp.int32, (RB, T), 0) + tb * RB
        c = jax.lax.broadcasted_iota(jnp.int32, (RB, T), 1)
        lt = (c < r).astype(f32)
        c_blocks.append(jax.lax.dot_general(lt, Epairs, (((1,), (0,)), ((), ())),
                                            preferred_element_type=f32))
    C = jnp.concatenate(c_blocks, axis=0)

    blk0 = jax.lax.dot_general(e0, excl, (((1,), (0,)), ((), ())),
                               preferred_element_type=f32)
    blk1 = jax.lax.dot_general(e1, excl, (((1,), (0,)), ((), ())),
                               preferred_element_type=f32)
    rank0 = jnp.sum(C * e0, axis=1, keepdims=True)
    rank1 = jnp.sum(C * e1, axis=1, keepdims=True)
    pos0 = blk0 * TB_ + rank0
    pos1 = blk1 * TB_ + rank1

    pos_ref[...] = jnp.concatenate([pos0, pos1], axis=1).astype(jnp.int32)

    cols = jnp.concatenate([pos0, pos1, w0, w1, w0, w0, w0, w0], axis=1)
    wc_ref[...] = cols
    ident = (jax.lax.broadcasted_iota(jnp.int32, (T, T), 0)
             == jax.lax.broadcasted_iota(jnp.int32, (T, T), 1)).astype(f32)
    pwt_ref[...] = jax.lax.dot_general(cols, ident, (((0,), (0,)), ((), ())),
                                       preferred_element_type=f32,
                                       precision=jax.lax.Precision.HIGHEST)

    bidx = jax.lax.broadcasted_iota(jnp.int32, (E, NBMAX), 1).astype(f32)
    bexp = jnp.sum((incl <= bidx).astype(f32), axis=0, keepdims=True)
    bexp = jnp.minimum(bexp, float(E - 1))
    nused = jnp.sum(nblk)
    nrow = jnp.full((1, NBMAX), nused, f32)
    bexp_ref[...] = jnp.concatenate(
        [bexp, nrow, bexp, bexp, bexp, bexp, bexp, bexp], axis=0
    ).astype(jnp.int32)


def _ffn_body(bexp_sref, num_sref, pwt_ref, h_ref, gu_ref, dn_ref, ys_ref,
              *, T, F, TB_):
    f32 = jnp.float32
    bf16 = jnp.bfloat16
    b = pl.program_id(0)

    @pl.when(b < num_sref[0])
    def _compute():
        pwt = pwt_ref[...]
        p0 = pwt[0:1, :]
        p1 = pwt[1:2, :]
        slot = (jax.lax.broadcasted_iota(jnp.int32, (TB_, T), 0)
                + b * TB_).astype(f32)
        Mg = (slot == p0).astype(bf16) + (slot == p1).astype(bf16)
        x = jax.lax.dot_general(Mg, h_ref[...].astype(bf16),
                                (((1,), (0,)), ((), ())),
                                preferred_element_type=f32)
        xb = x.astype(bf16)
        gu = gu_ref[0].astype(bf16)
        gate = jax.lax.dot_general(xb, gu[0:F], (((1,), (1,)), ((), ())),
                                   preferred_element_type=f32)
        up = jax.lax.dot_general(xb, gu[F:2 * F], (((1,), (1,)), ((), ())),
                                 preferred_element_type=f32)
        act = gate * (1.0 / (1.0 + jnp.exp(-gate))) * up
        ys_ref[...] = jax.lax.dot_general(act.astype(bf16),
                                          dn_ref[0].astype(bf16),
                                          (((1,), (1,)), ((), ())),
                                          preferred_element_type=f32)

def _make_combine_gather(NPAD, T, H):
    NW = SC_NC * SC_NS
    CH = T // NW
    mesh = plsc.VectorSubcoreMesh(core_axis_name="c", subcore_axis_name="s")

    @functools.partial(
        pl.kernel, mesh=mesh,
        out_type=(
            jax.ShapeDtypeStruct((T, H), jnp.float32),
            jax.ShapeDtypeStruct((T, H), jnp.float32),
        ),
        scratch_types=[
            pltpu.VMEM((CH,), jnp.int32),
            pltpu.VMEM((CH, H), jnp.float32),
            pltpu.SemaphoreType.DMA,
        ],
    )
    def combine(ys_hbm, p0_hbm, p1_hbm, yg0_hbm, yg1_hbm, idx_v, rows_v, sem):
        wid = lax.axis_index("s") * SC_NC + lax.axis_index("c")
        base = wid * CH
        pltpu.sync_copy(p0_hbm.at[pl.ds(base, CH)], idx_v)
        pltpu.async_copy(ys_hbm.at[idx_v], rows_v, sem).wait()
        pltpu.sync_copy(rows_v, yg0_hbm.at[pl.ds(base, CH)])
        pltpu.sync_copy(p1_hbm.at[pl.ds(base, CH)], idx_v)
        pltpu.async_copy(ys_hbm.at[idx_v], rows_v, sem).wait()
        pltpu.sync_copy(rows_v, yg1_hbm.at[pl.ds(base, CH)])

    return combine


def _combine_body(yg0_ref, yg1_ref, wc_ref, out_ref):
    wc = wc_ref[...]
    out_ref[...] = wc[:, 2:3] * yg0_ref[...] + wc[:, 3:4] * yg1_ref[...]


def kernel(hidden_states, router_weight, gate_up_proj, down_proj):
    Bv, Tv, Hv = hidden_states.shape
    E, H = router_weight.shape
    F = down_proj.shape[2]
    T = Bv * Tv
    K = 2
    NBMAX = T * K // TB + E - 1
    NBMAX = ((NBMAX + 7) // 8) * 8
    NPAD = NBMAX * TB

    h2 = hidden_states.reshape(T, Hv)

    pwt, bexp8, pos2, wc = pl.pallas_call(
        functools.partial(_router_body, T=T, E=E, TB_=TB, NBMAX=NBMAX),
        out_shape=(
            jax.ShapeDtypeStruct((8, T), jnp.float32),
            jax.ShapeDtypeStruct((8, NBMAX), jnp.int32),
            jax.ShapeDtypeStruct((T, 2), jnp.int32),
            jax.ShapeDtypeStruct((T, 8), jnp.float32),
        ),
    )(h2, router_weight)

    bexp = bexp8[0]
    num = bexp8[1, 0:1]

    grid_spec = pltpu.PrefetchScalarGridSpec(
        num_scalar_prefetch=2,
        grid=(NBMAX,),
        in_specs=[
            pl.BlockSpec((8, T), lambda b, be, n: (0, 0)),
            pl.BlockSpec((T, Hv), lambda b, be, n: (0, 0)),
            pl.BlockSpec((1, 2 * F, H), lambda b, be, n: (be[b], 0, 0)),
            pl.BlockSpec((1, H, F), lambda b, be, n: (be[b], 0, 0)),
        ],
        out_specs=pl.BlockSpec(
            (TB, Hv), lambda b, be, n: (jnp.minimum(b, n[0] - 1), 0)),
    )
    ys = pl.pallas_call(
        functools.partial(_ffn_body, T=T, F=F, TB_=TB),
        grid_spec=grid_spec,
        out_shape=jax.ShapeDtypeStruct((NPAD, Hv), jnp.float32),
        compiler_params=pltpu.CompilerParams(
            dimension_semantics=("arbitrary",)),
    )(bexp, num, pwt, h2, gate_up_proj, down_proj)

    p0 = pos2[:, 0]
    p1 = pos2[:, 1]
    yg0, yg1 = _make_combine_gather(NPAD, T, Hv)(ys, p0, p1)

    out = pl.pallas_call(
        _combine_body,
        out_shape=jax.ShapeDtypeStruct((T, Hv), jnp.float32),
    )(yg0, yg1, wc)

    return out.reshape(Bv, Tv, Hv)
